# Initial kernel scaffold; baseline (speedup 1.0000x reference)
import jax
import jax.numpy as jnp
from jax.experimental import pallas as pl


def kernel(x, edge_index, norm_A, W1, b1, gamma1, beta1, W2, b2, Wp, bp):
    raise NotImplementedError("write your pallas kernel here")


# speedup vs baseline: 2.3253x; 2.3253x over previous
import functools

import jax
import jax.numpy as jnp
from jax import lax
from jax.experimental import pallas as pl
from jax.experimental.pallas import tpu as pltpu
from jax.experimental.pallas import tpu_sc as plsc

N = 10000
E = 320000
D = 128
D_OUT = 40

NC = 2
NS = 16
NT = NC * NS
CHUNK = 128
G = 8
NW = 10
CPT = NW * G
EPT = CPT * CHUNK
EP = NT * EPT
NP = 10240
ROWS_PER_TILE = NP // NS


def _sc_segment_sum(h, src_r, dst_r, norm_r):
    mesh = plsc.VectorSubcoreMesh(core_axis_name="c", subcore_axis_name="s")

    @functools.partial(
        pl.kernel,
        out_type=jax.ShapeDtypeStruct((NC, NP, D), jnp.float32),
        mesh=mesh,
        scratch_types=[
            pltpu.VMEM((EPT,), jnp.int32),
            pltpu.VMEM((G, CHUNK), jnp.int32),
            pltpu.VMEM((G, CHUNK), jnp.float32),
            pltpu.VMEM((CHUNK, D), jnp.float32),
            pltpu.VMEM_SHARED((NP, D), jnp.float32),
            pltpu.SemaphoreType.DMA,
        ],
        compiler_params=pltpu.CompilerParams(needs_layout_passes=False),
    )
    def seg_sum(h_hbm, src_hbm, dst_hbm, norm_hbm, out_hbm,
                src_v, dst_w, norm_w, rows, acc_sh, sem):
        cid = lax.axis_index("c")
        sid = lax.axis_index("s")
        tid = cid * NS + sid

        pltpu.sync_copy(src_hbm.at[tid], src_v)

        zero16 = jnp.zeros((16,), jnp.float32)

        @pl.loop(0, CHUNK)
        def _(r):
            for q in range(D // 16):
                rows[r, pl.ds(q * 16, 16)] = zero16

        base = sid * ROWS_PER_TILE
        for i in range(ROWS_PER_TILE // CHUNK):
            pltpu.sync_copy(rows, acc_sh.at[pl.ds(base + i * CHUNK, CHUNK)])
        plsc.subcore_barrier()

        @pl.loop(0, NW)
        def _(w):
            pltpu.sync_copy(dst_hbm.at[tid, w], dst_w)
            pltpu.sync_copy(norm_hbm.at[tid, w], norm_w)

            @pl.loop(0, G)
            def _(g):
                j = w * G + g
                pltpu.async_copy(
                    h_hbm.at[src_v.at[pl.ds(j * CHUNK, CHUNK)]], rows, sem
                ).wait()

                @pl.loop(0, CHUNK)
                def _(r):
                    gg = lax.broadcast(g, (16,))
                    rr = lax.broadcast(r, (16,))
                    wgt = plsc.load_gather(norm_w, [gg, rr])
                    for q in range(D // 16):
                        sl = pl.ds(q * 16, 16)
                        rows[r, sl] = rows[r, sl] * wgt

                pltpu.sync_copy(rows, acc_sh.at[dst_w.at[g]], add=True)

        plsc.subcore_barrier()
        pltpu.sync_copy(acc_sh.at[pl.ds(base, ROWS_PER_TILE)],
                        out_hbm.at[cid, pl.ds(base, ROWS_PER_TILE)])

    return seg_sum(h, src_r, dst_r, norm_r)


def _tc_mm1(xp, W1):
    def body(x_ref, w_ref, o_ref):
        o_ref[...] = jnp.dot(x_ref[...], w_ref[...],
                             preferred_element_type=jnp.float32,
                             precision=lax.Precision.HIGHEST)

    return pl.pallas_call(
        body, out_shape=jax.ShapeDtypeStruct((NP, D), jnp.float32))(xp, W1)


def _tc_bn_relu_mm(p, gamma1, beta1, W2):
    def body(p_ref, g_ref, be_ref, w_ref, xs0_ref, h2_ref):
        ps = p_ref[0] + p_ref[1]
        s1 = jnp.sum(ps, axis=0)
        s2 = jnp.sum(ps * ps, axis=0)
        mean = s1 / N
        var = s2 / N - mean * mean
        inv = lax.rsqrt(var + 1e-5)
        xs0 = jnp.maximum(g_ref[...] * (ps - mean) * inv + be_ref[...], 0.0)
        xs0_ref[...] = xs0
        h2_ref[...] = jnp.dot(xs0, w_ref[...],
                              preferred_element_type=jnp.float32,
                              precision=lax.Precision.HIGHEST)

    return pl.pallas_call(
        body,
        out_shape=[jax.ShapeDtypeStruct((NP, D), jnp.float32),
                   jax.ShapeDtypeStruct((NP, D), jnp.float32)],
    )(p, gamma1, beta1, W2)


def _tc_final(xs0, q, b2, Wp_pad, bp_pad):
    def body(xs0_ref, q_ref, b2_ref, wp_ref, bp_ref, o_ref):
        xs1 = q_ref[0] + q_ref[1] + b2_ref[...]
        jk = jnp.maximum(xs0_ref[...], xs1)
        out = jnp.dot(jk, wp_ref[...],
                      preferred_element_type=jnp.float32,
                      precision=lax.Precision.HIGHEST) + bp_ref[...]
        col = lax.broadcasted_iota(jnp.int32, (NP, D), 1)
        z = jnp.where(col < D_OUT, out, -jnp.inf)
        m = jnp.max(z, axis=1, keepdims=True)
        lse = jnp.log(jnp.sum(jnp.exp(z - m), axis=1, keepdims=True)) + m
        o_ref[...] = out - lse

    return pl.pallas_call(
        body, out_shape=jax.ShapeDtypeStruct((NP, D), jnp.float32),
    )(xs0, q, b2, Wp_pad, bp_pad)


def kernel(x, edge_index, norm_A, W1, b1, gamma1, beta1, W2, b2, Wp, bp):
    src = edge_index[0]
    dst = edge_index[1]
    pad_e = EP - E
    src_r = jnp.concatenate(
        [src, jnp.zeros((pad_e,), jnp.int32)]).reshape(NT, EPT)
    dst_r = jnp.concatenate(
        [dst, jnp.zeros((pad_e,), jnp.int32)]).reshape(NT, NW, G, CHUNK)
    norm_r = jnp.concatenate(
        [norm_A, jnp.zeros((pad_e,), jnp.float32)]).reshape(NT, NW, G, CHUNK)
    xp = jnp.pad(x, ((0, NP - N), (0, 0)))
    Wp_pad = jnp.pad(Wp, ((0, 0), (0, D - D_OUT)))
    bp_pad = jnp.pad(bp, (0, D - D_OUT))

    h1 = _tc_mm1(xp, W1)
    p = _sc_segment_sum(h1, src_r, dst_r, norm_r)
    xs0, h2 = _tc_bn_relu_mm(p, gamma1, beta1, W2)
    q = _sc_segment_sum(h2, src_r, dst_r, norm_r)
    out = _tc_final(xs0, q, b2, Wp_pad, bp_pad)
    return out[:N, :D_OUT]
